# Initial kernel scaffold; baseline (speedup 1.0000x reference)
import jax
import jax.numpy as jnp
from jax.experimental import pallas as pl


def kernel(feat, y, centers):
    raise NotImplementedError("write your pallas kernel here")


# speedup vs baseline: 2.1888x; 2.1888x over previous
import functools

import jax
import jax.numpy as jnp
from jax import lax
from jax.experimental import pallas as pl
from jax.experimental.pallas import tpu as pltpu
from jax.experimental.pallas import tpu_sc as plsc

_L = 16
_NW = 32
_CLS_PAD = 1024


def _worker_id():
    return lax.axis_index("s") * 2 + lax.axis_index("c")


def _make_hist(batch):
    rows_per_w = batch // _NW
    n_batches = rows_per_w // 128
    mesh = plsc.VectorSubcoreMesh(core_axis_name="c", subcore_axis_name="s")

    @functools.partial(
        pl.kernel,
        mesh=mesh,
        out_type=jax.ShapeDtypeStruct((2 * _CLS_PAD, 128), jnp.float32),
        scratch_types=[
            pltpu.VMEM((rows_per_w,), jnp.int32),
            pltpu.VMEM((128,), jnp.int32),
            pltpu.VMEM((128, 128), jnp.float32),
            pltpu.VMEM_SHARED((_CLS_PAD, 128), jnp.float32),
        ],
    )
    def kern(y_hbm, zeros_hbm, ones_hbm, out_hbm, y_v, idx_v, ones_v, table):
        cid = lax.axis_index("c")
        sid = lax.axis_index("s")
        wid = sid * 2 + cid
        pltpu.sync_copy(y_hbm.at[pl.ds(wid * rows_per_w, rows_per_w)], y_v)
        pltpu.sync_copy(ones_hbm, ones_v)

        @pl.when(sid == 0)
        def _():
            pltpu.sync_copy(zeros_hbm, table)

        plsc.subcore_barrier()
        for b in range(n_batches):
            def off_step(i, carry):
                idx_v[pl.ds(i * _L, _L)] = y_v[pl.ds(b * 128 + i * _L, _L)]
                return carry
            lax.fori_loop(0, 128 // _L, off_step, 0)
            pltpu.sync_copy(ones_v, table.at[idx_v], add=True)
        plsc.subcore_barrier()

        @pl.when(sid == 0)
        def _():
            pltpu.sync_copy(table, out_hbm.at[pl.ds(cid * _CLS_PAD, _CLS_PAD)])

    return kern


def _make_combine():
    per_w = _CLS_PAD // _NW
    mesh = plsc.VectorSubcoreMesh(core_axis_name="c", subcore_axis_name="s")

    @functools.partial(
        pl.kernel,
        mesh=mesh,
        out_type=jax.ShapeDtypeStruct((_CLS_PAD, 128), jnp.float32),
        scratch_types=[
            pltpu.VMEM((2 * per_w, 128), jnp.float32),
            pltpu.VMEM((per_w, 128), jnp.float32),
        ],
    )
    def kern(parts_hbm, out_hbm, pbuf, inv_v):
        wid = _worker_id()
        for j in range(2):
            pltpu.sync_copy(
                parts_hbm.at[pl.ds(j * _CLS_PAD + wid * per_w, per_w)],
                pbuf.at[pl.ds(j * per_w, per_w)],
            )

        def cls_step(c, carry):
            acc = pbuf[c, pl.ds(0, _L)] + pbuf[per_w + c, pl.ds(0, _L)]
            invc = 1.0 / (acc + 1.0)
            for k in range(128 // _L):
                inv_v[c, pl.ds(k * _L, _L)] = invc
            return carry

        lax.fori_loop(0, per_w, cls_step, 0)
        pltpu.sync_copy(inv_v, out_hbm.at[pl.ds(wid * per_w, per_w)])

    return kern


def _make_loss(batch, feat_dim):
    rows_per_w = batch // _NW
    chunk = 32
    n_chunks = rows_per_w // chunk
    cols = feat_dim // _L
    mesh = plsc.VectorSubcoreMesh(core_axis_name="c", subcore_axis_name="s")

    @functools.partial(
        pl.kernel,
        mesh=mesh,
        out_type=jax.ShapeDtypeStruct((_NW, _L), jnp.float32),
        scratch_types=[
            pltpu.VMEM((chunk,), jnp.int32),
            pltpu.VMEM((chunk * feat_dim,), jnp.float32),
            pltpu.VMEM((chunk, feat_dim), jnp.float32),
            pltpu.VMEM((chunk, 128), jnp.float32),
            pltpu.VMEM((_L,), jnp.float32),
            pltpu.SemaphoreType.DMA,
            pltpu.SemaphoreType.DMA,
        ],
    )
    def kern(featf_hbm, y_hbm, centers_hbm, invtab_hbm, out_hbm,
             idx_v, frows, crows, winv, tot_v, sem_c, sem_w):
        wid = _worker_id()
        zeros = jnp.zeros((_L,), jnp.float32)

        def chunk_step(kc, total):
            base = wid * rows_per_w + kc * chunk
            pltpu.sync_copy(y_hbm.at[pl.ds(base, chunk)], idx_v)
            cdma = pltpu.async_copy(centers_hbm.at[idx_v], crows, sem_c)
            wdma = pltpu.async_copy(invtab_hbm.at[idx_v], winv, sem_w)
            pltpu.sync_copy(
                featf_hbm.at[pl.ds(base * feat_dim, chunk * feat_dim)], frows)
            cdma.wait()
            wdma.wait()

            def row_step(r, tot):
                def col_step(k, acc):
                    f = frows[pl.ds(r * feat_dim + k * _L, _L)]
                    ce = crows[r, pl.ds(k * _L, _L)]
                    d = f - ce
                    return acc + d * d

                acc = lax.fori_loop(0, cols, col_step, zeros)
                w16 = winv[r, pl.ds(0, _L)]
                return tot + w16 * acc

            return lax.fori_loop(0, chunk, row_step, total)

        total = lax.fori_loop(0, n_chunks, chunk_step, zeros)
        tot_v[...] = total
        pltpu.sync_copy(tot_v, out_hbm.at[wid])

    return kern


@jax.jit
def kernel(feat, y, centers):
    batch, feat_dim = feat.shape
    y32 = y.astype(jnp.int32)
    zeros_tab = jnp.zeros((_CLS_PAD, 128), jnp.float32)
    ones_blk = jnp.ones((128, 128), jnp.float32)
    parts = _make_hist(batch)(y32, zeros_tab, ones_blk)
    inv_tab = _make_combine()(parts)
    partials = _make_loss(batch, feat_dim)(
        feat.reshape(-1), y32, centers, inv_tab)
    return 0.5 * jnp.sum(partials)


# speedup vs baseline: 2.4836x; 1.1347x over previous
import functools

import jax
import jax.numpy as jnp
from jax import lax
from jax.experimental import pallas as pl
from jax.experimental.pallas import tpu as pltpu
from jax.experimental.pallas import tpu_sc as plsc

_L = 16
_NW = 32
_CLS_PAD = 1024
_SUB = 4


def _worker_id():
    return lax.axis_index("s") * 2 + lax.axis_index("c")


def _make_accum(batch, feat_dim):
    rows_per_w = batch // _NW
    chunk = 32
    blk_rows = 128
    n_blocks = rows_per_w // blk_rows
    sub_chunks = blk_rows // chunk
    s_rows = _CLS_PAD * _SUB
    mesh = plsc.VectorSubcoreMesh(core_axis_name="c", subcore_axis_name="s")

    @functools.partial(
        pl.kernel,
        mesh=mesh,
        out_type=[
            jax.ShapeDtypeStruct((2 * s_rows, 128), jnp.float32),
            jax.ShapeDtypeStruct((2 * _CLS_PAD, 128), jnp.float32),
        ],
        scratch_types=[
            pltpu.VMEM((rows_per_w,), jnp.int32),
            pltpu.VMEM((_SUB * rows_per_w,), jnp.int32),
            pltpu.VMEM((chunk * _SUB, 128), jnp.float32),
            pltpu.VMEM((blk_rows, 128), jnp.float32),
            pltpu.VMEM((chunk * _SUB,), jnp.int32),
            pltpu.VMEM((blk_rows,), jnp.int32),
            pltpu.VMEM_SHARED((s_rows, 128), jnp.float32),
            pltpu.VMEM_SHARED((_CLS_PAD, 128), jnp.float32),
        ],
    )
    def kern(feat_hbm, y_hbm, zeros_hbm, idx4_hbm, s_out, q_out,
             y_v, idx4_v, frows, qrows, sidx_v, qidx_v, s_tab, q_tab):
        cid = lax.axis_index("c")
        sid = lax.axis_index("s")
        wid = sid * 2 + cid
        pltpu.sync_copy(y_hbm.at[pl.ds(wid * rows_per_w, rows_per_w)], y_v)
        pltpu.sync_copy(
            idx4_hbm.at[pl.ds(wid * rows_per_w * _SUB, rows_per_w * _SUB)],
            idx4_v)

        zeros = jnp.zeros((_L,), jnp.float32)
        ones = jnp.ones((_L,), jnp.float32)
        for r in range(blk_rows):
            qrows[r, pl.ds(_L, _L)] = ones
            for k in range(2, 128 // _L):
                qrows[r, pl.ds(k * _L, _L)] = zeros

        @pl.when(sid == 0)
        def _():
            pltpu.sync_copy(zeros_hbm, s_tab)
            pltpu.sync_copy(zeros_hbm.at[pl.ds(0, _CLS_PAD)], q_tab)

        plsc.subcore_barrier()

        def blk_step(blk, carry):
            for i in range(blk_rows // _L):
                qidx_v[pl.ds(i * _L, _L)] = y_v[
                    pl.ds(blk * blk_rows + i * _L, _L)]

            def sub_step(c4, carry2):
                base = blk * blk_rows + c4 * chunk
                pltpu.sync_copy(
                    feat_hbm.at[pl.ds((wid * rows_per_w + base) * _SUB,
                                      chunk * _SUB)],
                    frows)
                for i in range(chunk * _SUB // _L):
                    sidx_v[pl.ds(i * _L, _L)] = idx4_v[
                        pl.ds(base * _SUB + i * _L, _L)]

                def row_step(r, carry3):
                    acc = zeros
                    for sub in range(_SUB):
                        for k in range(128 // _L):
                            f = frows[_SUB * r + sub, pl.ds(k * _L, _L)]
                            acc = acc + f * f
                    qrows[c4 * chunk + r, pl.ds(0, _L)] = acc
                    return carry3

                lax.fori_loop(0, chunk, row_step, 0)
                pltpu.sync_copy(frows, s_tab.at[sidx_v], add=True)
                return carry2

            lax.fori_loop(0, sub_chunks, sub_step, 0)
            pltpu.sync_copy(qrows, q_tab.at[qidx_v], add=True)
            return carry

        lax.fori_loop(0, n_blocks, blk_step, 0)
        plsc.subcore_barrier()

        @pl.when(sid == 0)
        def _():
            pltpu.sync_copy(s_tab, s_out.at[pl.ds(cid * s_rows, s_rows)])
            pltpu.sync_copy(q_tab, q_out.at[pl.ds(cid * _CLS_PAD, _CLS_PAD)])

    return kern


def _make_combine(feat_dim):
    per_w = _CLS_PAD // _NW
    s_rows = _CLS_PAD * _SUB
    mesh = plsc.VectorSubcoreMesh(core_axis_name="c", subcore_axis_name="s")

    @functools.partial(
        pl.kernel,
        mesh=mesh,
        out_type=jax.ShapeDtypeStruct((_NW, _L), jnp.float32),
        scratch_types=[
            pltpu.VMEM((2 * per_w * _SUB, 128), jnp.float32),
            pltpu.VMEM((2 * per_w, 128), jnp.float32),
            pltpu.VMEM((per_w, feat_dim), jnp.float32),
            pltpu.VMEM((_L,), jnp.float32),
        ],
    )
    def kern(s_hbm, q_hbm, cen_hbm, out_hbm, sbuf, qbuf, cbuf, tot_v):
        wid = _worker_id()
        for j in range(2):
            pltpu.sync_copy(
                s_hbm.at[pl.ds(j * s_rows + wid * per_w * _SUB, per_w * _SUB)],
                sbuf.at[pl.ds(j * per_w * _SUB, per_w * _SUB)],
            )
            pltpu.sync_copy(
                q_hbm.at[pl.ds(j * _CLS_PAD + wid * per_w, per_w)],
                qbuf.at[pl.ds(j * per_w, per_w)],
            )
        pltpu.sync_copy(cen_hbm.at[pl.ds(wid * per_w, per_w)], cbuf)

        zeros = jnp.zeros((_L,), jnp.float32)

        def cls_step(c, total):
            n16 = qbuf[c, pl.ds(_L, _L)] + qbuf[per_w + c, pl.ds(_L, _L)]
            inv16 = 1.0 / (n16 + 1.0)
            q16 = qbuf[c, pl.ds(0, _L)] + qbuf[per_w + c, pl.ds(0, _L)]
            dp = zeros
            cp = zeros
            for sub in range(_SUB):
                for k in range(128 // _L):
                    ce = cbuf[c, pl.ds(sub * 128 + k * _L, _L)]
                    sc = (sbuf[_SUB * c + sub, pl.ds(k * _L, _L)]
                          + sbuf[per_w * _SUB + _SUB * c + sub,
                                 pl.ds(k * _L, _L)])
                    dp = dp + sc * ce
                    cp = cp + ce * ce
            return total + inv16 * (q16 - 2.0 * dp + n16 * cp)

        total = lax.fori_loop(0, per_w, cls_step, zeros)
        tot_v[...] = total
        pltpu.sync_copy(tot_v, out_hbm.at[wid])

    return kern


@jax.jit
def kernel(feat, y, centers):
    batch, feat_dim = feat.shape
    n_cls = centers.shape[0]
    y32 = y.astype(jnp.int32)
    feat4 = feat.reshape(-1, 128)
    zeros_tab = jnp.zeros((_CLS_PAD * _SUB, 128), jnp.float32)
    cen_pad = jnp.pad(centers, ((0, _CLS_PAD - n_cls), (0, 0)))
    sub = jnp.arange(_SUB, dtype=jnp.int32)
    idx4 = (y32[:, None] * _SUB + sub[None, :]).reshape(-1)
    s_parts, q_parts = _make_accum(batch, feat_dim)(
        feat4, y32, zeros_tab, idx4)
    partials = _make_combine(feat_dim)(s_parts, q_parts, cen_pad)
    return 0.5 * jnp.sum(partials)


# speedup vs baseline: 2.7533x; 1.1086x over previous
import functools

import jax
import jax.numpy as jnp
from jax import lax
from jax.experimental import pallas as pl
from jax.experimental.pallas import tpu as pltpu
from jax.experimental.pallas import tpu_sc as plsc

_L = 16
_NW = 32
_CLS_PAD = 1024
_SUB = 4


def _worker_id():
    return lax.axis_index("s") * 2 + lax.axis_index("c")


def _make_accum(batch, feat_dim):
    rows_per_w = batch // _NW
    chunk = 32
    blk_rows = 128
    n_blocks = rows_per_w // blk_rows
    sub_chunks = blk_rows // chunk
    mesh = plsc.VectorSubcoreMesh(core_axis_name="c", subcore_axis_name="s")

    @functools.partial(
        pl.kernel,
        mesh=mesh,
        out_type=[
            jax.ShapeDtypeStruct((2 * _CLS_PAD, _SUB, 128), jnp.float32),
            jax.ShapeDtypeStruct((2 * _CLS_PAD, 128), jnp.float32),
        ],
        scratch_types=[
            pltpu.VMEM((rows_per_w,), jnp.int32),
            pltpu.VMEM((chunk, _SUB, 128), jnp.float32),
            pltpu.VMEM((blk_rows, 128), jnp.float32),
            pltpu.VMEM((chunk,), jnp.int32),
            pltpu.VMEM((blk_rows,), jnp.int32),
            pltpu.VMEM_SHARED((_CLS_PAD, _SUB, 128), jnp.float32),
            pltpu.VMEM_SHARED((_CLS_PAD, 128), jnp.float32),
        ],
    )
    def kern(feat_hbm, y_hbm, zs_hbm, zq_hbm, s_out, q_out,
             y_v, frows, qrows, sidx_v, qidx_v, s_tab, q_tab):
        cid = lax.axis_index("c")
        sid = lax.axis_index("s")
        wid = sid * 2 + cid
        pltpu.sync_copy(y_hbm.at[pl.ds(wid * rows_per_w, rows_per_w)], y_v)

        zeros = jnp.zeros((_L,), jnp.float32)
        ones = jnp.ones((_L,), jnp.float32)
        for r in range(blk_rows):
            qrows[r, pl.ds(_L, _L)] = ones
            for k in range(2, 128 // _L):
                qrows[r, pl.ds(k * _L, _L)] = zeros

        @pl.when(sid == 0)
        def _():
            pltpu.sync_copy(zs_hbm, s_tab)
            pltpu.sync_copy(zq_hbm, q_tab)

        plsc.subcore_barrier()

        def blk_step(blk, carry):
            for i in range(blk_rows // _L):
                qidx_v[pl.ds(i * _L, _L)] = y_v[
                    pl.ds(blk * blk_rows + i * _L, _L)]

            def sub_step(c4, carry2):
                base = blk * blk_rows + c4 * chunk
                pltpu.sync_copy(
                    feat_hbm.at[pl.ds(wid * rows_per_w + base, chunk)],
                    frows)
                for i in range(chunk // _L):
                    sidx_v[pl.ds(i * _L, _L)] = y_v[
                        pl.ds(base + i * _L, _L)]

                def row_step(r, carry3):
                    acc = zeros
                    for sub in range(_SUB):
                        for k in range(128 // _L):
                            f = frows[r, sub, pl.ds(k * _L, _L)]
                            acc = acc + f * f
                    qrows[c4 * chunk + r, pl.ds(0, _L)] = acc
                    return carry3

                lax.fori_loop(0, chunk, row_step, 0)
                pltpu.sync_copy(frows, s_tab.at[sidx_v], add=True)
                return carry2

            lax.fori_loop(0, sub_chunks, sub_step, 0)
            pltpu.sync_copy(qrows, q_tab.at[qidx_v], add=True)
            return carry

        lax.fori_loop(0, n_blocks, blk_step, 0)
        plsc.subcore_barrier()

        @pl.when(sid == 0)
        def _():
            pltpu.sync_copy(s_tab, s_out.at[pl.ds(cid * _CLS_PAD, _CLS_PAD)])
            pltpu.sync_copy(q_tab, q_out.at[pl.ds(cid * _CLS_PAD, _CLS_PAD)])

    return kern


def _make_combine(feat_dim):
    per_w = _CLS_PAD // _NW
    mesh = plsc.VectorSubcoreMesh(core_axis_name="c", subcore_axis_name="s")

    @functools.partial(
        pl.kernel,
        mesh=mesh,
        out_type=jax.ShapeDtypeStruct((_NW, _L), jnp.float32),
        scratch_types=[
            pltpu.VMEM((2 * per_w, _SUB, 128), jnp.float32),
            pltpu.VMEM((2 * per_w, 128), jnp.float32),
            pltpu.VMEM((per_w, feat_dim), jnp.float32),
            pltpu.VMEM((_L,), jnp.float32),
        ],
    )
    def kern(s_hbm, q_hbm, cen_hbm, out_hbm, sbuf, qbuf, cbuf, tot_v):
        wid = _worker_id()
        for j in range(2):
            pltpu.sync_copy(
                s_hbm.at[pl.ds(j * _CLS_PAD + wid * per_w, per_w)],
                sbuf.at[pl.ds(j * per_w, per_w)],
            )
            pltpu.sync_copy(
                q_hbm.at[pl.ds(j * _CLS_PAD + wid * per_w, per_w)],
                qbuf.at[pl.ds(j * per_w, per_w)],
            )
        pltpu.sync_copy(cen_hbm.at[pl.ds(wid * per_w, per_w)], cbuf)

        zeros = jnp.zeros((_L,), jnp.float32)

        def cls_step(c, total):
            n16 = qbuf[c, pl.ds(_L, _L)] + qbuf[per_w + c, pl.ds(_L, _L)]
            inv16 = 1.0 / (n16 + 1.0)
            q16 = qbuf[c, pl.ds(0, _L)] + qbuf[per_w + c, pl.ds(0, _L)]
            dp = zeros
            cp = zeros
            for sub in range(_SUB):
                for k in range(128 // _L):
                    ce = cbuf[c, pl.ds(sub * 128 + k * _L, _L)]
                    sc = (sbuf[c, sub, pl.ds(k * _L, _L)]
                          + sbuf[per_w + c, sub, pl.ds(k * _L, _L)])
                    dp = dp + sc * ce
                    cp = cp + ce * ce
            return total + inv16 * (q16 - 2.0 * dp + n16 * cp)

        total = lax.fori_loop(0, per_w, cls_step, zeros)
        tot_v[...] = total
        pltpu.sync_copy(tot_v, out_hbm.at[wid])

    return kern


@jax.jit
def kernel(feat, y, centers):
    batch, feat_dim = feat.shape
    n_cls = centers.shape[0]
    y32 = y.astype(jnp.int32)
    feat4 = feat.reshape(batch, _SUB, 128)
    zeros_s = jnp.zeros((_CLS_PAD, _SUB, 128), jnp.float32)
    zeros_q = jnp.zeros((_CLS_PAD, 128), jnp.float32)
    cen_pad = jnp.pad(centers, ((0, _CLS_PAD - n_cls), (0, 0)))
    s_parts, q_parts = _make_accum(batch, feat_dim)(
        feat4, y32, zeros_s, zeros_q)
    partials = _make_combine(feat_dim)(s_parts, q_parts, cen_pad)
    return 0.5 * jnp.sum(partials)


# speedup vs baseline: 2.7703x; 1.0062x over previous
import functools

import jax
import jax.numpy as jnp
from jax import lax
from jax.experimental import pallas as pl
from jax.experimental.pallas import tpu as pltpu
from jax.experimental.pallas import tpu_sc as plsc

_L = 16
_NW = 32
_CLS_PAD = 1024
_SUB = 4


def _worker_id():
    return lax.axis_index("s") * 2 + lax.axis_index("c")


def _make_accum(batch, feat_dim, chain):
    rows_per_w = batch // _NW
    chunk = 32
    blk_rows = 128
    n_blocks = rows_per_w // blk_rows
    sub_chunks = blk_rows // chunk
    s_seg = _CLS_PAD // 16
    q_seg = _CLS_PAD // 16
    mesh = plsc.VectorSubcoreMesh(core_axis_name="c", subcore_axis_name="s")

    @functools.partial(
        pl.kernel,
        mesh=mesh,
        out_type=[
            jax.ShapeDtypeStruct((2 * _CLS_PAD, _SUB, 128), jnp.float32),
            jax.ShapeDtypeStruct((2 * _CLS_PAD, 128), jnp.float32),
        ],
        scratch_types=[
            pltpu.VMEM((rows_per_w,), jnp.int32),
            pltpu.VMEM((chunk, _SUB, 128), jnp.float32),
            pltpu.VMEM((blk_rows, 128), jnp.float32),
            pltpu.VMEM((chunk,), jnp.int32),
            pltpu.VMEM((blk_rows,), jnp.int32),
            pltpu.VMEM_SHARED((_CLS_PAD, _SUB, 128), jnp.float32),
            pltpu.VMEM_SHARED((_CLS_PAD, 128), jnp.float32),
        ],
    )
    def kern(feat_hbm, y_hbm, zs_hbm, zq_hbm, s_out, q_out,
             y_v, frows, qrows, sidx_v, qidx_v, s_tab, q_tab):
        cid = lax.axis_index("c")
        sid = lax.axis_index("s")
        wid = sid * 2 + cid
        pltpu.sync_copy(y_hbm.at[pl.ds(wid * rows_per_w, rows_per_w)], y_v)

        zeros = jnp.zeros((_L,), jnp.float32)
        ones = jnp.ones((_L,), jnp.float32)
        for r in range(blk_rows):
            qrows[r, pl.ds(_L, _L)] = ones
            for k in range(2, 128 // _L):
                qrows[r, pl.ds(k * _L, _L)] = zeros

        zoff = cid * _CLS_PAD if chain else 0
        pltpu.sync_copy(
            zs_hbm.at[pl.ds(zoff + sid * s_seg, s_seg)],
            s_tab.at[pl.ds(sid * s_seg, s_seg)])
        pltpu.sync_copy(
            zq_hbm.at[pl.ds(zoff + sid * q_seg, q_seg)],
            q_tab.at[pl.ds(sid * q_seg, q_seg)])

        plsc.subcore_barrier()

        def blk_step(blk, carry):
            for i in range(blk_rows // _L):
                qidx_v[pl.ds(i * _L, _L)] = y_v[
                    pl.ds(blk * blk_rows + i * _L, _L)]

            def sub_step(c4, carry2):
                base = blk * blk_rows + c4 * chunk
                pltpu.sync_copy(
                    feat_hbm.at[pl.ds(wid * rows_per_w + base, chunk)],
                    frows)
                for i in range(chunk // _L):
                    sidx_v[pl.ds(i * _L, _L)] = y_v[
                        pl.ds(base + i * _L, _L)]

                def row_step(r, carry3):
                    acc = zeros
                    for sub in range(_SUB):
                        for k in range(128 // _L):
                            f = frows[r, sub, pl.ds(k * _L, _L)]
                            acc = acc + f * f
                    qrows[c4 * chunk + r, pl.ds(0, _L)] = acc
                    return carry3

                lax.fori_loop(0, chunk, row_step, 0)
                pltpu.sync_copy(frows, s_tab.at[sidx_v], add=True)
                return carry2

            lax.fori_loop(0, sub_chunks, sub_step, 0)
            pltpu.sync_copy(qrows, q_tab.at[qidx_v], add=True)
            return carry

        lax.fori_loop(0, n_blocks, blk_step, 0)
        plsc.subcore_barrier()

        @pl.when(sid == 0)
        def _():
            pltpu.sync_copy(s_tab, s_out.at[pl.ds(cid * _CLS_PAD, _CLS_PAD)])
            pltpu.sync_copy(q_tab, q_out.at[pl.ds(cid * _CLS_PAD, _CLS_PAD)])

    return kern


def _make_combine(feat_dim):
    per_w = _CLS_PAD // _NW
    mesh = plsc.VectorSubcoreMesh(core_axis_name="c", subcore_axis_name="s")

    @functools.partial(
        pl.kernel,
        mesh=mesh,
        out_type=jax.ShapeDtypeStruct((_NW, _L), jnp.float32),
        scratch_types=[
            pltpu.VMEM((2 * per_w, _SUB, 128), jnp.float32),
            pltpu.VMEM((2 * per_w, 128), jnp.float32),
            pltpu.VMEM((per_w, feat_dim), jnp.float32),
            pltpu.VMEM((_L,), jnp.float32),
        ],
    )
    def kern(s_hbm, q_hbm, cen_hbm, out_hbm, sbuf, qbuf, cbuf, tot_v):
        wid = _worker_id()
        for j in range(2):
            pltpu.sync_copy(
                s_hbm.at[pl.ds(j * _CLS_PAD + wid * per_w, per_w)],
                sbuf.at[pl.ds(j * per_w, per_w)],
            )
            pltpu.sync_copy(
                q_hbm.at[pl.ds(j * _CLS_PAD + wid * per_w, per_w)],
                qbuf.at[pl.ds(j * per_w, per_w)],
            )
        pltpu.sync_copy(cen_hbm.at[pl.ds(wid * per_w, per_w)], cbuf)

        zeros = jnp.zeros((_L,), jnp.float32)

        def cls_step(c, total):
            n16 = qbuf[c, pl.ds(_L, _L)] + qbuf[per_w + c, pl.ds(_L, _L)]
            inv16 = 1.0 / (n16 + 1.0)
            q16 = qbuf[c, pl.ds(0, _L)] + qbuf[per_w + c, pl.ds(0, _L)]
            dp = zeros
            cp = zeros
            for sub in range(_SUB):
                for k in range(128 // _L):
                    ce = cbuf[c, pl.ds(sub * 128 + k * _L, _L)]
                    sc = (sbuf[c, sub, pl.ds(k * _L, _L)]
                          + sbuf[per_w + c, sub, pl.ds(k * _L, _L)])
                    dp = dp + sc * ce
                    cp = cp + ce * ce
            return total + inv16 * (q16 - 2.0 * dp + n16 * cp)

        total = lax.fori_loop(0, per_w, cls_step, zeros)
        tot_v[...] = total
        pltpu.sync_copy(tot_v, out_hbm.at[wid])

    return kern


@jax.jit
def kernel(feat, y, centers):
    batch, feat_dim = feat.shape
    n_cls = centers.shape[0]
    y32 = y.astype(jnp.int32)
    feat4 = feat.reshape(batch, _SUB, 128)
    zeros_s = jnp.zeros((_CLS_PAD, _SUB, 128), jnp.float32)
    zeros_q = jnp.zeros((_CLS_PAD, 128), jnp.float32)
    cen_pad = jnp.pad(centers, ((0, _CLS_PAD - n_cls), (0, 0)))
    s_parts, q_parts = _make_accum(batch, feat_dim, chain=False)(
        feat4, y32, zeros_s, zeros_q)
    partials = _make_combine(feat_dim)(s_parts, q_parts, cen_pad)
    return 0.5 * jnp.sum(partials)


# speedup vs baseline: 2.8722x; 1.0368x over previous
import functools

import jax
import jax.numpy as jnp
from jax import lax
from jax.experimental import pallas as pl
from jax.experimental.pallas import tpu as pltpu
from jax.experimental.pallas import tpu_sc as plsc

_L = 16
_NW = 32
_CLS_PAD = 1024
_SUB = 4


def _worker_id():
    return lax.axis_index("s") * 2 + lax.axis_index("c")


def _make_accum(batch, feat_dim, chain):
    rows_per_w = batch // _NW
    chunk = 64
    blk_rows = 128
    n_blocks = rows_per_w // blk_rows
    sub_chunks = blk_rows // chunk
    s_seg = _CLS_PAD // 16
    q_seg = _CLS_PAD // 16
    mesh = plsc.VectorSubcoreMesh(core_axis_name="c", subcore_axis_name="s")

    @functools.partial(
        pl.kernel,
        mesh=mesh,
        out_type=[
            jax.ShapeDtypeStruct((2 * _CLS_PAD, _SUB, 128), jnp.float32),
            jax.ShapeDtypeStruct((2 * _CLS_PAD, 128), jnp.float32),
        ],
        scratch_types=[
            pltpu.VMEM((rows_per_w,), jnp.int32),
            pltpu.VMEM((chunk, _SUB, 128), jnp.float32),
            pltpu.VMEM((blk_rows, 128), jnp.float32),
            pltpu.VMEM((chunk,), jnp.int32),
            pltpu.VMEM((blk_rows,), jnp.int32),
            pltpu.VMEM_SHARED((_CLS_PAD, _SUB, 128), jnp.float32),
            pltpu.VMEM_SHARED((_CLS_PAD, 128), jnp.float32),
        ],
    )
    def kern(feat_hbm, y_hbm, zs_hbm, zq_hbm, s_out, q_out,
             y_v, frows, qrows, sidx_v, qidx_v, s_tab, q_tab):
        cid = lax.axis_index("c")
        sid = lax.axis_index("s")
        wid = sid * 2 + cid
        pltpu.sync_copy(y_hbm.at[pl.ds(wid * rows_per_w, rows_per_w)], y_v)

        zeros = jnp.zeros((_L,), jnp.float32)
        ones = jnp.ones((_L,), jnp.float32)
        for r in range(blk_rows):
            qrows[r, pl.ds(_L, _L)] = ones
            for k in range(2, 128 // _L):
                qrows[r, pl.ds(k * _L, _L)] = zeros

        zoff = cid * _CLS_PAD if chain else 0
        pltpu.sync_copy(
            zs_hbm.at[pl.ds(zoff + sid * s_seg, s_seg)],
            s_tab.at[pl.ds(sid * s_seg, s_seg)])
        pltpu.sync_copy(
            zq_hbm.at[pl.ds(zoff + sid * q_seg, q_seg)],
            q_tab.at[pl.ds(sid * q_seg, q_seg)])

        plsc.subcore_barrier()

        def blk_step(blk, carry):
            for i in range(blk_rows // _L):
                qidx_v[pl.ds(i * _L, _L)] = y_v[
                    pl.ds(blk * blk_rows + i * _L, _L)]

            def sub_step(c4, carry2):
                base = blk * blk_rows + c4 * chunk
                pltpu.sync_copy(
                    feat_hbm.at[pl.ds(wid * rows_per_w + base, chunk)],
                    frows)
                for i in range(chunk // _L):
                    sidx_v[pl.ds(i * _L, _L)] = y_v[
                        pl.ds(base + i * _L, _L)]

                def row_step(r, carry3):
                    acc = zeros
                    for sub in range(_SUB):
                        for k in range(128 // _L):
                            f = frows[r, sub, pl.ds(k * _L, _L)]
                            acc = acc + f * f
                    qrows[c4 * chunk + r, pl.ds(0, _L)] = acc
                    return carry3

                lax.fori_loop(0, chunk, row_step, 0)
                pltpu.sync_copy(frows, s_tab.at[sidx_v], add=True)
                return carry2

            lax.fori_loop(0, sub_chunks, sub_step, 0)
            pltpu.sync_copy(qrows, q_tab.at[qidx_v], add=True)
            return carry

        lax.fori_loop(0, n_blocks, blk_step, 0)
        plsc.subcore_barrier()

        @pl.when(sid == 0)
        def _():
            pltpu.sync_copy(s_tab, s_out.at[pl.ds(cid * _CLS_PAD, _CLS_PAD)])
            pltpu.sync_copy(q_tab, q_out.at[pl.ds(cid * _CLS_PAD, _CLS_PAD)])

    return kern


def _make_combine(feat_dim):
    per_w = _CLS_PAD // _NW
    mesh = plsc.VectorSubcoreMesh(core_axis_name="c", subcore_axis_name="s")

    @functools.partial(
        pl.kernel,
        mesh=mesh,
        out_type=jax.ShapeDtypeStruct((_NW, _L), jnp.float32),
        scratch_types=[
            pltpu.VMEM((2 * per_w, _SUB, 128), jnp.float32),
            pltpu.VMEM((2 * per_w, 128), jnp.float32),
            pltpu.VMEM((per_w, feat_dim), jnp.float32),
            pltpu.VMEM((_L,), jnp.float32),
        ],
    )
    def kern(s_hbm, q_hbm, cen_hbm, out_hbm, sbuf, qbuf, cbuf, tot_v):
        wid = _worker_id()
        for j in range(2):
            pltpu.sync_copy(
                s_hbm.at[pl.ds(j * _CLS_PAD + wid * per_w, per_w)],
                sbuf.at[pl.ds(j * per_w, per_w)],
            )
            pltpu.sync_copy(
                q_hbm.at[pl.ds(j * _CLS_PAD + wid * per_w, per_w)],
                qbuf.at[pl.ds(j * per_w, per_w)],
            )
        pltpu.sync_copy(cen_hbm.at[pl.ds(wid * per_w, per_w)], cbuf)

        zeros = jnp.zeros((_L,), jnp.float32)

        def cls_step(c, total):
            n16 = qbuf[c, pl.ds(_L, _L)] + qbuf[per_w + c, pl.ds(_L, _L)]
            inv16 = 1.0 / (n16 + 1.0)
            q16 = qbuf[c, pl.ds(0, _L)] + qbuf[per_w + c, pl.ds(0, _L)]
            dp = zeros
            cp = zeros
            for sub in range(_SUB):
                for k in range(128 // _L):
                    ce = cbuf[c, pl.ds(sub * 128 + k * _L, _L)]
                    sc = (sbuf[c, sub, pl.ds(k * _L, _L)]
                          + sbuf[per_w + c, sub, pl.ds(k * _L, _L)])
                    dp = dp + sc * ce
                    cp = cp + ce * ce
            return total + inv16 * (q16 - 2.0 * dp + n16 * cp)

        total = lax.fori_loop(0, per_w, cls_step, zeros)
        tot_v[...] = total
        pltpu.sync_copy(tot_v, out_hbm.at[wid])

    return kern


@jax.jit
def kernel(feat, y, centers):
    batch, feat_dim = feat.shape
    n_cls = centers.shape[0]
    y32 = y.astype(jnp.int32)
    feat4 = feat.reshape(batch, _SUB, 128)
    zeros_s = jnp.zeros((_CLS_PAD, _SUB, 128), jnp.float32)
    zeros_q = jnp.zeros((_CLS_PAD, 128), jnp.float32)
    cen_pad = jnp.pad(centers, ((0, _CLS_PAD - n_cls), (0, 0)))
    s_parts, q_parts = _make_accum(batch, feat_dim, chain=False)(
        feat4, y32, zeros_s, zeros_q)
    partials = _make_combine(feat_dim)(s_parts, q_parts, cen_pad)
    return 0.5 * jnp.sum(partials)


# speedup vs baseline: 2.9233x; 1.0178x over previous
import functools

import jax
import jax.numpy as jnp
from jax import lax
from jax.experimental import pallas as pl
from jax.experimental.pallas import tpu as pltpu
from jax.experimental.pallas import tpu_sc as plsc

_L = 16
_NW = 32
_CLS_PAD = 1024
_SUB = 4


def _worker_id():
    return lax.axis_index("s") * 2 + lax.axis_index("c")


def _make_accum(batch, feat_dim, chain):
    rows_per_w = batch // _NW
    chunk = 128
    blk_rows = 128
    n_blocks = rows_per_w // blk_rows
    sub_chunks = blk_rows // chunk
    s_seg = _CLS_PAD // 16
    q_seg = _CLS_PAD // 16
    mesh = plsc.VectorSubcoreMesh(core_axis_name="c", subcore_axis_name="s")

    @functools.partial(
        pl.kernel,
        mesh=mesh,
        out_type=[
            jax.ShapeDtypeStruct((2 * _CLS_PAD, _SUB, 128), jnp.float32),
            jax.ShapeDtypeStruct((2 * _CLS_PAD, 128), jnp.float32),
        ],
        scratch_types=[
            pltpu.VMEM((rows_per_w,), jnp.int32),
            pltpu.VMEM((chunk, _SUB, 128), jnp.float32),
            pltpu.VMEM((blk_rows, 128), jnp.float32),
            pltpu.VMEM((chunk,), jnp.int32),
            pltpu.VMEM((blk_rows,), jnp.int32),
            pltpu.VMEM_SHARED((_CLS_PAD, _SUB, 128), jnp.float32),
            pltpu.VMEM_SHARED((_CLS_PAD, 128), jnp.float32),
        ],
    )
    def kern(feat_hbm, y_hbm, zs_hbm, zq_hbm, s_out, q_out,
             y_v, frows, qrows, sidx_v, qidx_v, s_tab, q_tab):
        cid = lax.axis_index("c")
        sid = lax.axis_index("s")
        wid = sid * 2 + cid
        pltpu.sync_copy(y_hbm.at[pl.ds(wid * rows_per_w, rows_per_w)], y_v)

        zeros = jnp.zeros((_L,), jnp.float32)
        ones = jnp.ones((_L,), jnp.float32)
        for r in range(blk_rows):
            qrows[r, pl.ds(_L, _L)] = ones
            for k in range(2, 128 // _L):
                qrows[r, pl.ds(k * _L, _L)] = zeros

        zoff = cid * _CLS_PAD if chain else 0
        pltpu.sync_copy(
            zs_hbm.at[pl.ds(zoff + sid * s_seg, s_seg)],
            s_tab.at[pl.ds(sid * s_seg, s_seg)])
        pltpu.sync_copy(
            zq_hbm.at[pl.ds(zoff + sid * q_seg, q_seg)],
            q_tab.at[pl.ds(sid * q_seg, q_seg)])

        plsc.subcore_barrier()

        def blk_step(blk, carry):
            for i in range(blk_rows // _L):
                qidx_v[pl.ds(i * _L, _L)] = y_v[
                    pl.ds(blk * blk_rows + i * _L, _L)]

            def sub_step(c4, carry2):
                base = blk * blk_rows + c4 * chunk
                pltpu.sync_copy(
                    feat_hbm.at[pl.ds(wid * rows_per_w + base, chunk)],
                    frows)
                for i in range(chunk // _L):
                    sidx_v[pl.ds(i * _L, _L)] = y_v[
                        pl.ds(base + i * _L, _L)]

                def row_step(r, carry3):
                    acc = zeros
                    for sub in range(_SUB):
                        for k in range(128 // _L):
                            f = frows[r, sub, pl.ds(k * _L, _L)]
                            acc = acc + f * f
                    qrows[c4 * chunk + r, pl.ds(0, _L)] = acc
                    return carry3

                lax.fori_loop(0, chunk, row_step, 0)
                pltpu.sync_copy(frows, s_tab.at[sidx_v], add=True)
                return carry2

            lax.fori_loop(0, sub_chunks, sub_step, 0)
            pltpu.sync_copy(qrows, q_tab.at[qidx_v], add=True)
            return carry

        lax.fori_loop(0, n_blocks, blk_step, 0)
        plsc.subcore_barrier()

        @pl.when(sid == 0)
        def _():
            pltpu.sync_copy(s_tab, s_out.at[pl.ds(cid * _CLS_PAD, _CLS_PAD)])
            pltpu.sync_copy(q_tab, q_out.at[pl.ds(cid * _CLS_PAD, _CLS_PAD)])

    return kern


def _make_combine(feat_dim):
    per_w = _CLS_PAD // _NW
    mesh = plsc.VectorSubcoreMesh(core_axis_name="c", subcore_axis_name="s")

    @functools.partial(
        pl.kernel,
        mesh=mesh,
        out_type=jax.ShapeDtypeStruct((_NW, _L), jnp.float32),
        scratch_types=[
            pltpu.VMEM((2 * per_w, _SUB, 128), jnp.float32),
            pltpu.VMEM((2 * per_w, 128), jnp.float32),
            pltpu.VMEM((per_w, feat_dim), jnp.float32),
            pltpu.VMEM((_L,), jnp.float32),
        ],
    )
    def kern(s_hbm, q_hbm, cen_hbm, out_hbm, sbuf, qbuf, cbuf, tot_v):
        wid = _worker_id()
        for j in range(2):
            pltpu.sync_copy(
                s_hbm.at[pl.ds(j * _CLS_PAD + wid * per_w, per_w)],
                sbuf.at[pl.ds(j * per_w, per_w)],
            )
            pltpu.sync_copy(
                q_hbm.at[pl.ds(j * _CLS_PAD + wid * per_w, per_w)],
                qbuf.at[pl.ds(j * per_w, per_w)],
            )
        pltpu.sync_copy(cen_hbm.at[pl.ds(wid * per_w, per_w)], cbuf)

        zeros = jnp.zeros((_L,), jnp.float32)

        def cls_step(c, total):
            n16 = qbuf[c, pl.ds(_L, _L)] + qbuf[per_w + c, pl.ds(_L, _L)]
            inv16 = 1.0 / (n16 + 1.0)
            q16 = qbuf[c, pl.ds(0, _L)] + qbuf[per_w + c, pl.ds(0, _L)]
            dp = zeros
            cp = zeros
            for sub in range(_SUB):
                for k in range(128 // _L):
                    ce = cbuf[c, pl.ds(sub * 128 + k * _L, _L)]
                    sc = (sbuf[c, sub, pl.ds(k * _L, _L)]
                          + sbuf[per_w + c, sub, pl.ds(k * _L, _L)])
                    dp = dp + sc * ce
                    cp = cp + ce * ce
            return total + inv16 * (q16 - 2.0 * dp + n16 * cp)

        total = lax.fori_loop(0, per_w, cls_step, zeros)
        tot_v[...] = total
        pltpu.sync_copy(tot_v, out_hbm.at[wid])

    return kern


@jax.jit
def kernel(feat, y, centers):
    batch, feat_dim = feat.shape
    n_cls = centers.shape[0]
    y32 = y.astype(jnp.int32)
    feat4 = feat.reshape(batch, _SUB, 128)
    zeros_s = jnp.zeros((_CLS_PAD, _SUB, 128), jnp.float32)
    zeros_q = jnp.zeros((_CLS_PAD, 128), jnp.float32)
    cen_pad = jnp.pad(centers, ((0, _CLS_PAD - n_cls), (0, 0)))
    s_parts, q_parts = _make_accum(batch, feat_dim, chain=False)(
        feat4, y32, zeros_s, zeros_q)
    partials = _make_combine(feat_dim)(s_parts, q_parts, cen_pad)
    return 0.5 * jnp.sum(partials)


# speedup vs baseline: 2.9246x; 1.0005x over previous
import functools

import jax
import jax.numpy as jnp
from jax import lax
from jax.experimental import pallas as pl
from jax.experimental.pallas import tpu as pltpu
from jax.experimental.pallas import tpu_sc as plsc

_L = 16
_NW = 32
_CLS_PAD = 1024
_SUB = 4


def _worker_id():
    return lax.axis_index("s") * 2 + lax.axis_index("c")


def _make_accum(batch, feat_dim):
    rows_per_w = batch // _NW
    chunk = 128
    blk_rows = 128
    n_blocks = rows_per_w // blk_rows
    sub_chunks = blk_rows // chunk
    s_seg = _CLS_PAD // 16
    q_seg = _CLS_PAD // 16
    mesh = plsc.VectorSubcoreMesh(core_axis_name="c", subcore_axis_name="s")

    @functools.partial(
        pl.kernel,
        mesh=mesh,
        out_type=[
            jax.ShapeDtypeStruct((2 * _CLS_PAD, _SUB, 128), jnp.float32),
            jax.ShapeDtypeStruct((2 * _CLS_PAD, 128), jnp.float32),
        ],
        scratch_types=[
            pltpu.VMEM((rows_per_w,), jnp.int32),
            pltpu.VMEM((chunk, _SUB, 128), jnp.float32),
            pltpu.VMEM((blk_rows, 128), jnp.float32),
            pltpu.VMEM((chunk,), jnp.int32),
            pltpu.VMEM((blk_rows,), jnp.int32),
            pltpu.VMEM_SHARED((_CLS_PAD, _SUB, 128), jnp.float32),
            pltpu.VMEM_SHARED((_CLS_PAD, 128), jnp.float32),
        ],
    )
    def kern(feat_hbm, y_hbm, zs_hbm, zq_hbm, s_out, q_out,
             y_v, frows, qrows, sidx_v, qidx_v, s_tab, q_tab):
        cid = lax.axis_index("c")
        sid = lax.axis_index("s")
        wid = sid * 2 + cid
        pltpu.sync_copy(y_hbm.at[pl.ds(wid * rows_per_w, rows_per_w)], y_v)

        zeros = jnp.zeros((_L,), jnp.float32)
        ones = jnp.ones((_L,), jnp.float32)
        for r in range(blk_rows):
            qrows[r, pl.ds(_L, _L)] = ones
            for k in range(2, 128 // _L):
                qrows[r, pl.ds(k * _L, _L)] = zeros

        pltpu.sync_copy(
            zs_hbm.at[pl.ds(sid * s_seg, s_seg)],
            s_tab.at[pl.ds(sid * s_seg, s_seg)])
        pltpu.sync_copy(
            zq_hbm.at[pl.ds(sid * q_seg, q_seg)],
            q_tab.at[pl.ds(sid * q_seg, q_seg)])

        plsc.subcore_barrier()

        def blk_step(blk, carry):
            for i in range(blk_rows // _L):
                qidx_v[pl.ds(i * _L, _L)] = y_v[
                    pl.ds(blk * blk_rows + i * _L, _L)]

            def sub_step(c4, carry2):
                base = blk * blk_rows + c4 * chunk
                pltpu.sync_copy(
                    feat_hbm.at[pl.ds(wid * rows_per_w + base, chunk)],
                    frows)
                for i in range(chunk // _L):
                    sidx_v[pl.ds(i * _L, _L)] = y_v[
                        pl.ds(base + i * _L, _L)]

                def row_step(r, carry3):
                    acc = zeros
                    for sub in range(_SUB):
                        for k in range(128 // _L):
                            f = frows[r, sub, pl.ds(k * _L, _L)]
                            acc = acc + f * f
                    qrows[c4 * chunk + r, pl.ds(0, _L)] = acc
                    return carry3

                lax.fori_loop(0, chunk, row_step, 0)
                pltpu.sync_copy(frows, s_tab.at[sidx_v], add=True)
                return carry2

            lax.fori_loop(0, sub_chunks, sub_step, 0)
            pltpu.sync_copy(qrows, q_tab.at[qidx_v], add=True)
            return carry

        lax.fori_loop(0, n_blocks, blk_step, 0)
        plsc.subcore_barrier()

        @pl.when(sid == 0)
        def _():
            pltpu.sync_copy(s_tab, s_out.at[pl.ds(cid * _CLS_PAD, _CLS_PAD)])
            pltpu.sync_copy(q_tab, q_out.at[pl.ds(cid * _CLS_PAD, _CLS_PAD)])

    return kern


def _make_combine(feat_dim):
    per_w = _CLS_PAD // _NW
    mesh = plsc.VectorSubcoreMesh(core_axis_name="c", subcore_axis_name="s")

    @functools.partial(
        pl.kernel,
        mesh=mesh,
        out_type=jax.ShapeDtypeStruct((_NW, _L), jnp.float32),
        scratch_types=[
            pltpu.VMEM((2 * per_w, _SUB, 128), jnp.float32),
            pltpu.VMEM((2 * per_w, 128), jnp.float32),
            pltpu.VMEM((per_w, feat_dim), jnp.float32),
            pltpu.VMEM((_L,), jnp.float32),
        ],
    )
    def kern(s_hbm, q_hbm, cen_hbm, out_hbm, sbuf, qbuf, cbuf, tot_v):
        wid = _worker_id()
        for j in range(2):
            pltpu.sync_copy(
                s_hbm.at[pl.ds(j * _CLS_PAD + wid * per_w, per_w)],
                sbuf.at[pl.ds(j * per_w, per_w)],
            )
            pltpu.sync_copy(
                q_hbm.at[pl.ds(j * _CLS_PAD + wid * per_w, per_w)],
                qbuf.at[pl.ds(j * per_w, per_w)],
            )
        pltpu.sync_copy(cen_hbm.at[pl.ds(wid * per_w, per_w)], cbuf)

        zeros = jnp.zeros((_L,), jnp.float32)

        def cls_step(c, total):
            n16 = qbuf[c, pl.ds(_L, _L)] + qbuf[per_w + c, pl.ds(_L, _L)]
            inv16 = 1.0 / (n16 + 1.0)
            q16 = qbuf[c, pl.ds(0, _L)] + qbuf[per_w + c, pl.ds(0, _L)]
            dp = zeros
            cp = zeros
            for sub in range(_SUB):
                for k in range(128 // _L):
                    ce = cbuf[c, pl.ds(sub * 128 + k * _L, _L)]
                    sc = (sbuf[c, sub, pl.ds(k * _L, _L)]
                          + sbuf[per_w + c, sub, pl.ds(k * _L, _L)])
                    dp = dp + sc * ce
                    cp = cp + ce * ce
            return total + inv16 * (q16 - 2.0 * dp + n16 * cp)

        total = lax.fori_loop(0, per_w, cls_step, zeros)
        tot_v[...] = total
        pltpu.sync_copy(tot_v, out_hbm.at[wid])

    return kern


@jax.jit
def kernel(feat, y, centers):
    batch, feat_dim = feat.shape
    n_cls = centers.shape[0]
    y32 = y.astype(jnp.int32)
    feat4 = feat.reshape(batch, _SUB, 128)
    zeros_s = jnp.zeros((_CLS_PAD, _SUB, 128), jnp.float32)
    zeros_q = jnp.zeros((_CLS_PAD, 128), jnp.float32)
    cen_pad = jnp.pad(centers, ((0, _CLS_PAD - n_cls), (0, 0)))
    s_parts, q_parts = _make_accum(batch, feat_dim)(
        feat4, y32, zeros_s, zeros_q)
    partials = _make_combine(feat_dim)(s_parts, q_parts, cen_pad)
    return 0.5 * jnp.sum(partials)
